# baseline (speedup 1.0000x reference)
import functools

import jax
import jax.numpy as jnp
from jax.experimental import pallas as pl
from jax.experimental.pallas import tpu as pltpu

N_NODES = 10000
N_PAD = 10240
BLK = 256


def _mm_relu_body(t_ref, w_ref, b_ref, o_ref):
    o_ref[...] = jax.nn.relu(
        jnp.dot(t_ref[...], w_ref[...], preferred_element_type=jnp.float32)
        + b_ref[...]
    )


def _mm_relu(tcat, wcat, b):
    n_pad, kdim = tcat.shape
    hid = wcat.shape[1]
    grid = n_pad // BLK
    return pl.pallas_call(
        _mm_relu_body,
        grid=(grid,),
        in_specs=[
            pl.BlockSpec((BLK, kdim), lambda i: (i, i * 0)),
            pl.BlockSpec((kdim, hid), lambda i: (i * 0, i * 0)),
            pl.BlockSpec((1, hid), lambda i: (i * 0, i * 0)),
        ],
        out_specs=pl.BlockSpec((BLK, hid), lambda i: (i, i * 0)),
        out_shape=jax.ShapeDtypeStruct((n_pad, hid), jnp.float32),
    )(tcat, wcat, b)


def _mm_relu_sum_body(t_ref, w_ref, b_ref, o_ref, *, n_valid):
    i = pl.program_id(0)
    h = jax.nn.relu(
        jnp.dot(t_ref[...], w_ref[...], preferred_element_type=jnp.float32)
        + b_ref[...]
    )
    row = i * BLK + jax.lax.broadcasted_iota(jnp.int32, (BLK, 1), 0)
    h = jnp.where(row < n_valid, h, 0.0)
    part = jnp.sum(h, axis=0, keepdims=True)

    @pl.when(i == 0)
    def _():
        o_ref[...] = jnp.zeros_like(o_ref)

    o_ref[...] += part


def _mm_relu_sum(tcat, wcat, b, n_valid):
    n_pad, kdim = tcat.shape
    hid = wcat.shape[1]
    grid = n_pad // BLK
    return pl.pallas_call(
        functools.partial(_mm_relu_sum_body, n_valid=n_valid),
        grid=(grid,),
        in_specs=[
            pl.BlockSpec((BLK, kdim), lambda i: (i, i * 0)),
            pl.BlockSpec((kdim, hid), lambda i: (i * 0, i * 0)),
            pl.BlockSpec((1, hid), lambda i: (i * 0, i * 0)),
        ],
        out_specs=pl.BlockSpec((1, hid), lambda i: (i * 0, i * 0)),
        out_shape=jax.ShapeDtypeStruct((1, hid), jnp.float32),
    )(tcat, wcat, b)


def _heads_body(s_ref, wmu_ref, bmu_ref, wlv_ref, blv_ref, mu_ref, lv_ref):
    ge = s_ref[...] * (1.0 / N_NODES)
    mu_ref[...] = (
        jnp.dot(ge, wmu_ref[...], preferred_element_type=jnp.float32) + bmu_ref[...]
    )
    lv_ref[...] = (
        jnp.dot(ge, wlv_ref[...], preferred_element_type=jnp.float32) + blv_ref[...]
    )


def _heads(hsum, Wmu, bmu, Wlv, blv):
    lat = Wmu.shape[1]
    return pl.pallas_call(
        _heads_body,
        out_shape=(
            jax.ShapeDtypeStruct((1, lat), jnp.float32),
            jax.ShapeDtypeStruct((1, lat), jnp.float32),
        ),
    )(hsum, Wmu, bmu.reshape(1, -1), Wlv, blv.reshape(1, -1))


def _pad_rows(a, n_pad):
    return jnp.pad(a, ((0, n_pad - a.shape[0]), (0, 0)))


def kernel(x, edge_index, lap_pe, edge_weight, W1, b1, W2, b2, Wmu, bmu, Wlv, blv):
    num_nodes = x.shape[0]
    src = edge_index[0].astype(jnp.int32)
    dst = edge_index[1].astype(jnp.int32)
    loop = jnp.arange(num_nodes, dtype=jnp.int32)
    src_e = jnp.concatenate([src, loop])
    dst_e = jnp.concatenate([dst, loop])
    w_e = jnp.concatenate([edge_weight, jnp.ones((num_nodes,), edge_weight.dtype)])
    deg = jax.ops.segment_sum(w_e, src_e, num_segments=num_nodes)
    dis = jnp.where(deg > 0, jax.lax.rsqrt(deg), 0.0)
    w_norm = -(dis[src_e] * w_e * dis[dst_e])

    def mv(t):
        return jax.ops.segment_sum(
            w_norm[:, None] * t[src_e], dst_e, num_segments=num_nodes
        )

    def cheb_txs(h, k):
        txs = [h]
        tx1 = mv(h)
        txs.append(tx1)
        tx0 = h
        for _ in range(2, k):
            tx2 = 2.0 * mv(tx1) - tx0
            txs.append(tx2)
            tx0, tx1 = tx1, tx2
        return txs

    x_comb = jnp.concatenate([x, lap_pe], axis=1)
    k = W1.shape[0]

    txs1 = cheb_txs(x_comb, k)
    tcat1 = _pad_rows(jnp.concatenate(txs1, axis=1), N_PAD)
    wcat1 = W1.reshape(k * W1.shape[1], W1.shape[2])
    h1 = _mm_relu(tcat1, wcat1, b1.reshape(1, -1))[:num_nodes]

    txs2 = cheb_txs(h1, k)
    tcat2 = _pad_rows(jnp.concatenate(txs2, axis=1), N_PAD)
    wcat2 = W2.reshape(k * W2.shape[1], W2.shape[2])
    hsum = _mm_relu_sum(tcat2, wcat2, b2.reshape(1, -1), num_nodes)

    mu, lv = _heads(hsum, Wmu, bmu, Wlv, blv)
    return (mu, lv)


# speedup vs baseline: 2.9928x; 2.9928x over previous
import functools

import jax
import jax.numpy as jnp
from jax import lax
from jax.experimental import pallas as pl
from jax.experimental.pallas import tpu as pltpu
from jax.experimental.pallas import tpu_sc as plsc

N_NODES = 10000
BLK = 400

NW = 32
EB = 128
NB = 81
EPT = NB * EB
E_PAD = NW * EPT
N_ACC = 10240
RPT = N_ACC // 16
DR = 128
SB_LEN = 27


def _make_mv(D):
    mesh = plsc.VectorSubcoreMesh(core_axis_name="c", subcore_axis_name="s")

    @functools.partial(
        pl.kernel,
        out_type=jax.ShapeDtypeStruct((2, N_ACC, D), jnp.float32),
        mesh=mesh,
        compiler_params=pltpu.CompilerParams(use_tc_tiling_on_sc=False),
        scratch_types=[
            pltpu.VMEM((SB_LEN, EB), jnp.int32),
            pltpu.VMEM((SB_LEN, EB), jnp.int32),
            pltpu.VMEM((SB_LEN, EB), jnp.float32),
            pltpu.VMEM((EB, D), jnp.float32),
            pltpu.VMEM_SHARED((N_ACC, D), jnp.float32),
            pltpu.SemaphoreType.DMA,
        ],
    )
    def mv(t_hbm, src_hbm, dst_hbm, w_hbm, out_hbm,
           srcv, dstv, wv, rows, acc, sem):
        c = lax.axis_index("c")
        s = lax.axis_index("s")
        wid = s * jnp.int32(2) + c

        def zrow(r, _):
            for ch in range(D // 16):
                rows[r, pl.ds(ch * 16, 16)] = jnp.zeros((16,), jnp.float32)
            return 0

        lax.fori_loop(jnp.int32(0), jnp.int32(EB), zrow, 0)
        base = s * jnp.int32(RPT)
        for k in range(RPT // DR):
            pltpu.sync_copy(rows, acc.at[pl.ds(base + jnp.int32(k * DR), DR)])
        plsc.subcore_barrier()

        def superstep(sb, _):
            off = sb * jnp.int32(SB_LEN)
            pltpu.sync_copy(src_hbm.at[wid, pl.ds(off, SB_LEN)], srcv)
            pltpu.sync_copy(dst_hbm.at[wid, pl.ds(off, SB_LEN)], dstv)
            pltpu.sync_copy(w_hbm.at[wid, pl.ds(off, SB_LEN)], wv)

            def step(g, _):
                pltpu.async_copy(t_hbm.at[srcv.at[g]], rows, sem).wait()

                def sgrp(q, _):
                    w16 = wv[g, pl.ds(q * jnp.int32(16), 16)]
                    for jj in range(16):
                        wj = w16[jj]
                        j = q * jnp.int32(16) + jnp.int32(jj)
                        for ch in range(D // 16):
                            sl = pl.ds(ch * 16, 16)
                            rows[j, sl] = rows[j, sl] * wj
                    return 0

                lax.fori_loop(jnp.int32(0), jnp.int32(EB // 16), sgrp, 0)
                pltpu.sync_copy(rows, acc.at[dstv.at[g]], add=True)
                return 0

            lax.fori_loop(jnp.int32(0), jnp.int32(SB_LEN), step, 0)
            return 0

        lax.fori_loop(jnp.int32(0), jnp.int32(NB // SB_LEN), superstep, 0)
        plsc.subcore_barrier()

        for k in range(RPT // DR):
            st = base + jnp.int32(k * DR)
            pltpu.sync_copy(acc.at[pl.ds(st, DR)], rows)
            pltpu.sync_copy(rows, out_hbm.at[c, pl.ds(st, DR)])

    return mv


_MV = {d: _make_mv(d) for d in (144, 128)}


def _comb2_body(p_ref, o_ref, *, alpha):
    o_ref[...] = alpha * (p_ref[0] + p_ref[1])


def _comb3_body(p_ref, prev_ref, o_ref, *, alpha):
    o_ref[...] = alpha * (p_ref[0] + p_ref[1]) - prev_ref[...]


def _combine(p, prev, alpha):
    d = p.shape[2]
    grid = N_NODES // BLK
    pspec = pl.BlockSpec((2, BLK, d), lambda i: (i * 0, i, i * 0))
    ospec = pl.BlockSpec((BLK, d), lambda i: (i, i * 0))
    oshape = jax.ShapeDtypeStruct((N_NODES, d), jnp.float32)
    if prev is None:
        return pl.pallas_call(
            functools.partial(_comb2_body, alpha=alpha),
            grid=(grid,), in_specs=[pspec], out_specs=ospec, out_shape=oshape,
        )(p)
    return pl.pallas_call(
        functools.partial(_comb3_body, alpha=alpha),
        grid=(grid,), in_specs=[pspec, ospec], out_specs=ospec, out_shape=oshape,
    )(p, prev)


def _mm_relu_body(t0, t1, t2, t3, w_ref, b_ref, o_ref):
    tcat = jnp.concatenate([t0[...], t1[...], t2[...], t3[...]], axis=1)
    o_ref[...] = jax.nn.relu(
        jnp.dot(tcat, w_ref[...], preferred_element_type=jnp.float32) + b_ref[...]
    )


def _mm_relu(txs, wcat, b):
    d = txs[0].shape[1]
    kdim, hid = wcat.shape
    grid = N_NODES // BLK
    tspec = pl.BlockSpec((BLK, d), lambda i: (i, i * 0))
    return pl.pallas_call(
        _mm_relu_body,
        grid=(grid,),
        in_specs=[tspec, tspec, tspec, tspec,
                  pl.BlockSpec((kdim, hid), lambda i: (i * 0, i * 0)),
                  pl.BlockSpec((1, hid), lambda i: (i * 0, i * 0))],
        out_specs=pl.BlockSpec((BLK, hid), lambda i: (i, i * 0)),
        out_shape=jax.ShapeDtypeStruct((N_NODES, hid), jnp.float32),
    )(*txs, wcat, b)


def _mm_relu_sum_body(t0, t1, t2, t3, w_ref, b_ref, o_ref):
    i = pl.program_id(0)
    tcat = jnp.concatenate([t0[...], t1[...], t2[...], t3[...]], axis=1)
    h = jax.nn.relu(
        jnp.dot(tcat, w_ref[...], preferred_element_type=jnp.float32) + b_ref[...]
    )
    part = jnp.sum(h, axis=0, keepdims=True)

    @pl.when(i == 0)
    def _():
        o_ref[...] = jnp.zeros_like(o_ref)

    o_ref[...] += part


def _mm_relu_sum(txs, wcat, b):
    d = txs[0].shape[1]
    kdim, hid = wcat.shape
    grid = N_NODES // BLK
    tspec = pl.BlockSpec((BLK, d), lambda i: (i, i * 0))
    return pl.pallas_call(
        _mm_relu_sum_body,
        grid=(grid,),
        in_specs=[tspec, tspec, tspec, tspec,
                  pl.BlockSpec((kdim, hid), lambda i: (i * 0, i * 0)),
                  pl.BlockSpec((1, hid), lambda i: (i * 0, i * 0))],
        out_specs=pl.BlockSpec((1, hid), lambda i: (i * 0, i * 0)),
        out_shape=jax.ShapeDtypeStruct((1, hid), jnp.float32),
    )(*txs, wcat, b)


def _heads_body(s_ref, wmu_ref, bmu_ref, wlv_ref, blv_ref, mu_ref, lv_ref):
    ge = s_ref[...] * (1.0 / N_NODES)
    mu_ref[...] = (
        jnp.dot(ge, wmu_ref[...], preferred_element_type=jnp.float32) + bmu_ref[...]
    )
    lv_ref[...] = (
        jnp.dot(ge, wlv_ref[...], preferred_element_type=jnp.float32) + blv_ref[...]
    )


def _heads(hsum, Wmu, bmu, Wlv, blv):
    lat = Wmu.shape[1]
    return pl.pallas_call(
        _heads_body,
        out_shape=(
            jax.ShapeDtypeStruct((1, lat), jnp.float32),
            jax.ShapeDtypeStruct((1, lat), jnp.float32),
        ),
    )(hsum, Wmu, bmu.reshape(1, -1), Wlv, blv.reshape(1, -1))


def kernel(x, edge_index, lap_pe, edge_weight, W1, b1, W2, b2, Wmu, bmu, Wlv, blv):
    num_nodes = x.shape[0]
    src = edge_index[0].astype(jnp.int32)
    dst = edge_index[1].astype(jnp.int32)
    loop = jnp.arange(num_nodes, dtype=jnp.int32)
    src_e = jnp.concatenate([src, loop])
    dst_e = jnp.concatenate([dst, loop])
    w_e = jnp.concatenate([edge_weight, jnp.ones((num_nodes,), edge_weight.dtype)])
    deg = jax.ops.segment_sum(w_e, src_e, num_segments=num_nodes)
    dis = jnp.where(deg > 0, lax.rsqrt(deg), 0.0)
    w_norm = -(dis[src_e] * w_e * dis[dst_e])

    ne = src_e.shape[0]
    pad = E_PAD - ne
    srcp = jnp.pad(src_e, (0, pad)).reshape(NW, NB, EB)
    dstp = jnp.pad(dst_e, (0, pad)).reshape(NW, NB, EB)
    wp = jnp.pad(w_norm.astype(jnp.float32), (0, pad)).reshape(NW, NB, EB)

    def mv(t):
        return _MV[t.shape[1]](t, srcp, dstp, wp)

    def cheb_txs(h):
        tx1 = _combine(mv(h), None, 1.0)
        tx2 = _combine(mv(tx1), h, 2.0)
        tx3 = _combine(mv(tx2), tx1, 2.0)
        return [h, tx1, tx2, tx3]

    x_comb = jnp.concatenate([x, lap_pe], axis=1)
    k = W1.shape[0]

    txs1 = cheb_txs(x_comb)
    wcat1 = W1.reshape(k * W1.shape[1], W1.shape[2])
    h1 = _mm_relu(txs1, wcat1, b1.reshape(1, -1))

    txs2 = cheb_txs(h1)
    wcat2 = W2.reshape(k * W2.shape[1], W2.shape[2])
    hsum = _mm_relu_sum(txs2, wcat2, b2.reshape(1, -1))

    mu, lv = _heads(hsum, Wmu, bmu, Wlv, blv)
    return (mu, lv)
